# baseline (speedup 1.0000x reference)
import jax
import jax.numpy as jnp
from jax import lax
from jax.experimental import pallas as pl
from jax.experimental.pallas import tpu as pltpu
from jax.experimental.pallas import tpu_sc as plsc

BATCH = 16384
DEPTH = 64
HIDDEN = 128
MARGIN = 1.0

N_ENT = 1_000_000
W = 512
NFULL = N_ENT // W
NBLK = NFULL + 1
TAIL_W = N_ENT - NFULL * W
TAIL_OFF = NFULL * W

KE = 4 * BATCH
KR = 2 * BATCH
NW = 32
BLK_ITERS = (NFULL + NW - 1) // NW
TAIL_WID = NFULL % NW

BB = 2048


def _vscalar(vecref, j):
    chunk = vecref[pl.ds(16 * lax.div(j, 16), 16)]
    lane = lax.iota(jnp.int32, 16)
    return jnp.sum(jnp.where(lane == lax.rem(j, 16), chunk, 0))


def _windows(bufref, off, k0, k1, se_h, sea_h, out_h,
             outst, ses, seas, ssem):
    m0 = lax.div(k0, 128)
    m1 = lax.div(k1 + 127, 128)

    def window_body(m, carry2):
        wb = m * 128
        pltpu.sync_copy(se_h.at[pl.ds(wb, 128)], ses)
        pltpu.sync_copy(sea_h.at[pl.ds(wb, 128)], seas)
        jlo = jnp.maximum(k0, wb)
        jhi = jnp.minimum(k1, wb + 128)

        def row_body(j, carry3):
            jl = j - wb
            e = _vscalar(ses, jl) - off
            orow = _vscalar(seas, jl)
            evec = jnp.full((16,), e, jnp.int32)
            rvec = jnp.full((16,), jl, jnp.int32)
            for g in range(4):
                dvec = lax.iota(jnp.int32, 16) + 16 * g
                vals = plsc.load_gather(bufref, [dvec, evec])
                plsc.store_scatter(outst, [rvec, dvec], vals)
            pltpu.async_copy(outst.at[jl], out_h.at[orow], ssem)
            return carry3

        lax.fori_loop(jlo, jhi, row_body, 0)

        def drain_body(i, carry3):
            pltpu.make_async_copy(outst.at[0], out_h.at[0], ssem).wait()
            return carry3

        lax.fori_loop(0, jhi - jlo, drain_body, 0)
        return carry2

    lax.fori_loop(m0, m1, window_body, 0)


def _load_ptrs(ptr_h, b, ptrs):
    pltpu.sync_copy(ptr_h.at[b], ptrs)
    pv = ptrs[...]
    lane = lax.iota(jnp.int32, 16)
    k0 = jnp.sum(jnp.where(lane == 0, pv, 0))
    k1 = jnp.sum(jnp.where(lane == 1, pv, 0))
    return k0, k1


def _extract_table(wid, tableT, tail_h, se_h, sea_h, ptr_h, out_h,
                   buf, tailbuf, outst, ses, seas, ptrs, ssem):
    def block_body(bi, carry):
        b = wid + NW * bi

        @pl.when(b < NFULL)
        def _():
            off = b * W
            pltpu.sync_copy(tableT.at[:, pl.ds(off, W)], buf)
            k0, k1 = _load_ptrs(ptr_h, b, ptrs)

            @pl.when(k1 > k0)
            def _():
                _windows(buf, off, k0, k1, se_h, sea_h, out_h,
                         outst, ses, seas, ssem)

        return carry

    lax.fori_loop(0, BLK_ITERS, block_body, 0)

    @pl.when(wid == TAIL_WID)
    def _():
        k0, k1 = _load_ptrs(ptr_h, NFULL, ptrs)

        @pl.when(k1 > k0)
        def _():
            pltpu.sync_copy(tail_h, tailbuf)
            _windows(tailbuf, TAIL_OFF, k0, k1, se_h, sea_h, out_h,
                     outst, ses, seas, ssem)


def _sc_body(entT, relT, etail, rtail, se_h, sea_h, eptr_h, sr_h, sra_h,
             rptr_h, ent_out, rel_out,
             buf, tailbuf, outst, ses, seas, ptrs, bsem, ssem):
    nc = plsc.get_sparse_core_info().num_cores
    wid = lax.axis_index("s") * nc + lax.axis_index("c")
    _extract_table(wid, entT, etail, se_h, sea_h, eptr_h, ent_out,
                   buf, tailbuf, outst, ses, seas, ptrs, ssem)
    _extract_table(wid, relT, rtail, sr_h, sra_h, rptr_h, rel_out,
                   buf, tailbuf, outst, ses, seas, ptrs, ssem)


def _sc_extract(entT, relT, etail, rtail, se, sea, eptr2, sr, sra, rptr2):
    k = pl.kernel(
        _sc_body,
        out_type=[
            jax.ShapeDtypeStruct((KE, DEPTH), jnp.float32),
            jax.ShapeDtypeStruct((KR, DEPTH), jnp.float32),
        ],
        mesh=plsc.VectorSubcoreMesh(core_axis_name="c", subcore_axis_name="s"),
        compiler_params=pltpu.CompilerParams(needs_layout_passes=False),
        scratch_types=[
            pltpu.VMEM((DEPTH, W), jnp.float32),
            pltpu.VMEM((DEPTH, TAIL_W), jnp.float32),
            pltpu.VMEM((128, DEPTH), jnp.float32),
            pltpu.VMEM((128,), jnp.int32),
            pltpu.VMEM((128,), jnp.int32),
            pltpu.VMEM((16,), jnp.int32),
            pltpu.SemaphoreType.DMA,
            pltpu.SemaphoreType.DMA,
        ],
    )
    return k(entT, relT, etail, rtail, se, sea, eptr2, sr, sra, rptr2)


def _tc_body(hp, tp, rp, hg, tg, rg, w1, b1, w2, b2, out_ref):
    i = pl.program_id(0)

    w1v = w1[...]
    w1a = w1v[:DEPTH]
    w1b = w1v[DEPTH:]
    w2v = w2[...]
    b1v = b1[...]
    b2v = b2[...]

    def normalize(x):
        ss = jnp.sum(x * x, axis=1, keepdims=True)
        return x / jnp.maximum(jnp.sqrt(ss), 1e-12)

    def score(h, t, r):
        hn = normalize(h)
        tn = normalize(t)
        hid = lax.dot_general(hn, w1a, (((1,), (0,)), ((), ())),
                              preferred_element_type=jnp.float32)
        hid += lax.dot_general(r, w1b, (((1,), (0,)), ((), ())),
                               preferred_element_type=jnp.float32)
        hid = jnp.maximum(hid + b1v, 0.0)
        out = lax.dot_general(hid, w2v, (((1,), (0,)), ((), ())),
                              preferred_element_type=jnp.float32) + b2v
        d = out - tn
        return jnp.sqrt(jnp.sum(d * d, axis=1))

    ps = score(hp[...], tp[...], rp[...])
    ns = score(hg[...], tg[...], rg[...])
    part = jnp.sum(jnp.maximum(MARGIN + ps - ns, 0.0)).reshape(1, 1)

    @pl.when(i == 0)
    def _():
        out_ref[...] = jnp.zeros((1, 1), jnp.float32)

    out_ref[...] += part

    @pl.when(i == pl.num_programs(0) - 1)
    def _():
        out_ref[...] = out_ref[...] * (1.0 / BATCH)


def _tc_score(ent_rows, rel_rows, W1, b1, W2, b2):
    nb = BATCH // BB
    grid = (nb,)
    row_spec = lambda off: pl.BlockSpec((BB, DEPTH), lambda i, o=off: (i + o, 0))
    res = pl.pallas_call(
        _tc_body,
        grid=grid,
        in_specs=[
            row_spec(0),
            row_spec(nb),
            pl.BlockSpec((BB, DEPTH), lambda i: (i, 0)),
            row_spec(2 * nb),
            row_spec(3 * nb),
            pl.BlockSpec((BB, DEPTH), lambda i: (i + nb, 0)),
            pl.BlockSpec((2 * DEPTH, HIDDEN), lambda i: (0, 0)),
            pl.BlockSpec((1, HIDDEN), lambda i: (0, 0)),
            pl.BlockSpec((HIDDEN, DEPTH), lambda i: (0, 0)),
            pl.BlockSpec((1, DEPTH), lambda i: (0, 0)),
        ],
        out_specs=pl.BlockSpec((1, 1), lambda i: (0, 0)),
        out_shape=jax.ShapeDtypeStruct((1, 1), jnp.float32),
    )(ent_rows, ent_rows, rel_rows, ent_rows, ent_rows, rel_rows,
      W1, b1.reshape(1, HIDDEN), W2, b2.reshape(1, DEPTH))
    return res[0, 0]


def _csr(sorted_vals, n_rows):
    starts = jnp.arange(NBLK + 1, dtype=jnp.int32) * W
    ptr = jnp.searchsorted(sorted_vals, starts, side="left").astype(jnp.int32)
    ptr = ptr.at[NBLK].set(n_rows)
    pad = jnp.zeros((NBLK, 14), jnp.int32)
    return jnp.concatenate(
        [ptr[:-1, None], ptr[1:, None], pad], axis=1)


def kernel(pos_x, neg_x, ent_table, rel_table, W1, b1, W2, b2):
    eidx = jnp.concatenate(
        [pos_x[:, 0], pos_x[:, 1], neg_x[:, 0], neg_x[:, 1]])
    ridx = jnp.concatenate([pos_x[:, 2], neg_x[:, 2]])

    sea = jnp.argsort(eidx).astype(jnp.int32)
    se = jnp.take(eidx, sea)
    sra = jnp.argsort(ridx).astype(jnp.int32)
    sr = jnp.take(ridx, sra)
    eptr2 = _csr(se, KE)
    rptr2 = _csr(sr, KR)

    entT = ent_table.T
    relT = rel_table.T
    etail = lax.slice(entT, (0, TAIL_OFF), (DEPTH, N_ENT))
    rtail = lax.slice(relT, (0, TAIL_OFF), (DEPTH, N_ENT))

    ent_rows, rel_rows = _sc_extract(
        entT, relT, etail, rtail, se, sea, eptr2, sr, sra, rptr2)
    return _tc_score(ent_rows, rel_rows, W1, b1, W2, b2)


# speedup vs baseline: 1.0919x; 1.0919x over previous
import jax
import jax.numpy as jnp
from jax import lax
from jax.experimental import pallas as pl
from jax.experimental.pallas import tpu as pltpu
from jax.experimental.pallas import tpu_sc as plsc

BATCH = 16384
DEPTH = 64
HIDDEN = 128
MARGIN = 1.0

N_ENT = 1_000_000
W = 512
NFULL = N_ENT // W
NBLK = NFULL + 1
TAIL_W = N_ENT - NFULL * W
TAIL_OFF = NFULL * W

KE = 4 * BATCH
KR = 2 * BATCH
NW = 32
BLK_ITERS = (NFULL + NW - 1) // NW
TAIL_WID = NFULL % NW

BB = 2048


def _windows(bufref, off, k0, k1, dummy, se_h, sea_h, out_h,
             outst, sev, seav, ssem):
    lane = lax.iota(jnp.int32, 16)
    m0 = lax.div(k0, 128)
    m1 = lax.div(k1 + 127, 128)

    def window_body(m, carry2):
        wb = m * 128
        pltpu.sync_copy(se_h.at[pl.ds(wb, 128)], sev)
        pltpu.sync_copy(sea_h.at[pl.ds(wb, 128)], seav)
        jlo = jnp.maximum(k0, wb)
        jhi = jnp.minimum(k1, wb + 128)
        qlo = lax.div(jlo - wb, 16)
        qhi = lax.div(jhi - wb + 15, 16)

        def chunk_body(q, carry3):
            kvec = wb + 16 * q + lane
            valid = (kvec >= k0) & (kvec < k1)
            sev16 = sev[pl.ds(16 * q, 16)]
            seav16 = seav[pl.ds(16 * q, 16)]
            evec = jnp.where(valid, sev16 - off, 0)
            idxv = jnp.where(valid, seav16, dummy)
            slotv = 16 * q + lane
            for d in range(DEPTH):
                dv = jnp.full((16,), d, jnp.int32)
                vals = plsc.load_gather(bufref, [dv, evec])
                plsc.store_scatter(outst, [slotv, dv], vals)
            pltpu.async_copy(outst.at[pl.ds(16 * q, 16)], out_h.at[idxv],
                             ssem)
            return carry3

        lax.fori_loop(qlo, qhi, chunk_body, 0)

        def drain_body(i, carry3):
            pltpu.make_async_copy(outst.at[pl.ds(0, 16)],
                                  out_h.at[pl.ds(0, 16)], ssem).wait()
            return carry3

        lax.fori_loop(0, qhi - qlo, drain_body, 0)
        return carry2

    lax.fori_loop(m0, m1, window_body, 0)


def _load_ptrs(ptr_h, b, ptrs):
    pltpu.sync_copy(ptr_h.at[b], ptrs)
    pv = ptrs[...]
    lane = lax.iota(jnp.int32, 16)
    k0 = jnp.sum(jnp.where(lane == 0, pv, 0))
    k1 = jnp.sum(jnp.where(lane == 1, pv, 0))
    return k0, k1


def _extract_table(wid, tableT, tail_h, se_h, sea_h, ptr_h, out_h, dummy,
                   buf0, buf1, tailbuf, outst, sev, seav, ptrs, bsem, ssem):
    bufs = (buf0, buf1)

    def start_load(bi, buf):
        b = wid + NW * bi

        @pl.when(b < NFULL)
        def _():
            pltpu.async_copy(tableT.at[:, pl.ds(b * W, W)], buf, bsem)

    start_load(0, bufs[0])

    def pair_body(it, carry):
        for par in range(2):
            bi = 2 * it + par
            b = wid + NW * bi

            @pl.when(b < NFULL)
            def _(par=par, b=b, bi=bi):
                pltpu.make_async_copy(tableT.at[:, pl.ds(0, W)], bufs[par],
                                      bsem).wait()
                start_load(bi + 1, bufs[1 - par])
                off = b * W
                k0, k1 = _load_ptrs(ptr_h, b, ptrs)

                @pl.when(k1 > k0)
                def _():
                    _windows(bufs[par], off, k0, k1, dummy, se_h, sea_h,
                             out_h, outst, sev, seav, ssem)

        return carry

    lax.fori_loop(0, (BLK_ITERS + 1) // 2, pair_body, 0)

    @pl.when(wid == TAIL_WID)
    def _():
        k0, k1 = _load_ptrs(ptr_h, NFULL, ptrs)

        @pl.when(k1 > k0)
        def _():
            pltpu.sync_copy(tail_h, tailbuf)
            _windows(tailbuf, TAIL_OFF, k0, k1, dummy, se_h, sea_h, out_h,
                     outst, sev, seav, ssem)


def _sc_body(entT, relT, etail, rtail, se_h, sea_h, eptr_h, sr_h, sra_h,
             rptr_h, ent_out, rel_out,
             buf0, buf1, tailbuf, outst, sev, seav, ptrs, bsem, ssem):
    nc = plsc.get_sparse_core_info().num_cores
    wid = lax.axis_index("s") * nc + lax.axis_index("c")
    dummy = KE + lax.rem(wid, 16)
    _extract_table(wid, entT, etail, se_h, sea_h, eptr_h, ent_out, dummy,
                   buf0, buf1, tailbuf, outst, sev, seav, ptrs, bsem, ssem)
    dummy_r = KR + lax.rem(wid, 16)
    _extract_table(wid, relT, rtail, sr_h, sra_h, rptr_h, rel_out, dummy_r,
                   buf0, buf1, tailbuf, outst, sev, seav, ptrs, bsem, ssem)


def _sc_extract(entT, relT, etail, rtail, se, sea, eptr2, sr, sra, rptr2):
    k = pl.kernel(
        _sc_body,
        out_type=[
            jax.ShapeDtypeStruct((KE + 16, 2 * DEPTH), jnp.float32),
            jax.ShapeDtypeStruct((KR + 16, 2 * DEPTH), jnp.float32),
        ],
        mesh=plsc.VectorSubcoreMesh(core_axis_name="c", subcore_axis_name="s"),
        compiler_params=pltpu.CompilerParams(needs_layout_passes=False),
        scratch_types=[
            pltpu.VMEM((DEPTH, W), jnp.float32),
            pltpu.VMEM((DEPTH, W), jnp.float32),
            pltpu.VMEM((DEPTH, TAIL_W), jnp.float32),
            pltpu.VMEM((128, 2 * DEPTH), jnp.float32),
            pltpu.VMEM((128,), jnp.int32),
            pltpu.VMEM((128,), jnp.int32),
            pltpu.VMEM((16,), jnp.int32),
            pltpu.SemaphoreType.DMA,
            pltpu.SemaphoreType.DMA,
        ],
    )
    return k(entT, relT, etail, rtail, se, sea, eptr2, sr, sra, rptr2)


def _tc_body(hp, tp, rp, hg, tg, rg, w1, b1, w2, b2, out_ref):
    i = pl.program_id(0)

    w1v = w1[...]
    w1a = w1v[:DEPTH]
    w1b = w1v[DEPTH:]
    w2v = w2[...]
    b1v = b1[...]
    b2v = b2[...]

    def normalize(x):
        ss = jnp.sum(x * x, axis=1, keepdims=True)
        return x / jnp.maximum(jnp.sqrt(ss), 1e-12)

    def score(h, t, r):
        hn = normalize(h)
        tn = normalize(t)
        hid = lax.dot_general(hn, w1a, (((1,), (0,)), ((), ())),
                              preferred_element_type=jnp.float32)
        hid += lax.dot_general(r, w1b, (((1,), (0,)), ((), ())),
                               preferred_element_type=jnp.float32)
        hid = jnp.maximum(hid + b1v, 0.0)
        out = lax.dot_general(hid, w2v, (((1,), (0,)), ((), ())),
                              preferred_element_type=jnp.float32) + b2v
        d = out - tn
        return jnp.sqrt(jnp.sum(d * d, axis=1))

    def half(ref):
        return ref[...][:, :DEPTH]

    ps = score(half(hp), half(tp), half(rp))
    ns = score(half(hg), half(tg), half(rg))
    part = jnp.sum(jnp.maximum(MARGIN + ps - ns, 0.0)).reshape(1, 1)

    @pl.when(i == 0)
    def _():
        out_ref[...] = jnp.zeros((1, 1), jnp.float32)

    out_ref[...] += part

    @pl.when(i == pl.num_programs(0) - 1)
    def _():
        out_ref[...] = out_ref[...] * (1.0 / BATCH)


def _tc_score(ent_rows, rel_rows, W1, b1, W2, b2):
    nb = BATCH // BB
    grid = (nb,)
    row_spec = lambda off: pl.BlockSpec(
        (BB, 2 * DEPTH), lambda i, o=off: (i + o, 0))
    res = pl.pallas_call(
        _tc_body,
        grid=grid,
        in_specs=[
            row_spec(0),
            row_spec(nb),
            pl.BlockSpec((BB, 2 * DEPTH), lambda i: (i, 0)),
            row_spec(2 * nb),
            row_spec(3 * nb),
            pl.BlockSpec((BB, 2 * DEPTH), lambda i: (i + nb, 0)),
            pl.BlockSpec((2 * DEPTH, HIDDEN), lambda i: (0, 0)),
            pl.BlockSpec((1, HIDDEN), lambda i: (0, 0)),
            pl.BlockSpec((HIDDEN, DEPTH), lambda i: (0, 0)),
            pl.BlockSpec((1, DEPTH), lambda i: (0, 0)),
        ],
        out_specs=pl.BlockSpec((1, 1), lambda i: (0, 0)),
        out_shape=jax.ShapeDtypeStruct((1, 1), jnp.float32),
    )(ent_rows, ent_rows, rel_rows, ent_rows, ent_rows, rel_rows,
      W1, b1.reshape(1, HIDDEN), W2, b2.reshape(1, DEPTH))
    return res[0, 0]


def _csr(sorted_vals, n_rows):
    starts = jnp.arange(NBLK + 1, dtype=jnp.int32) * W
    ptr = jnp.searchsorted(sorted_vals, starts, side="left").astype(jnp.int32)
    ptr = ptr.at[NBLK].set(n_rows)
    pad = jnp.zeros((NBLK, 14), jnp.int32)
    return jnp.concatenate(
        [ptr[:-1, None], ptr[1:, None], pad], axis=1)


def kernel(pos_x, neg_x, ent_table, rel_table, W1, b1, W2, b2):
    eidx = jnp.concatenate(
        [pos_x[:, 0], pos_x[:, 1], neg_x[:, 0], neg_x[:, 1]])
    ridx = jnp.concatenate([pos_x[:, 2], neg_x[:, 2]])

    sea = jnp.argsort(eidx).astype(jnp.int32)
    se = jnp.take(eidx, sea)
    sra = jnp.argsort(ridx).astype(jnp.int32)
    sr = jnp.take(ridx, sra)
    eptr2 = _csr(se, KE)
    rptr2 = _csr(sr, KR)

    entT = ent_table.T
    relT = rel_table.T
    etail = lax.slice(entT, (0, TAIL_OFF), (DEPTH, N_ENT))
    rtail = lax.slice(relT, (0, TAIL_OFF), (DEPTH, N_ENT))

    ent_rows, rel_rows = _sc_extract(
        entT, relT, etail, rtail, se, sea, eptr2, sr, sra, rptr2)
    return _tc_score(ent_rows, rel_rows, W1, b1, W2, b2)
